# baseline (speedup 1.0000x reference)
import functools

import jax
import jax.numpy as jnp
from jax import lax
from jax.experimental import pallas as pl
from jax.experimental.pallas import tpu as pltpu
from jax.experimental.pallas import tpu_sc as plsc

N_NODES = 10000
N_EDGES = 320000
D = 128

NC = 2
NS = 16
NW = NC * NS
EPW = N_EDGES // NW

CH = 80
N_CHUNKS = EPW // CH

CH_DEG = 200
N_CHUNKS_DEG = EPW // CH_DEG


def _drain(dummy_hbm_slice, dst_ref, sem):
    pltpu.make_async_copy(dummy_hbm_slice, dst_ref, sem).wait()

ROWS_PER_TILE = 624
TAIL_ROWS = N_NODES - NS * ROWS_PER_TILE
TAIL_OFF = NS * ROWS_PER_TILE

_mesh = plsc.VectorSubcoreMesh(core_axis_name="c", subcore_axis_name="s")


@functools.partial(
    pl.kernel,
    mesh=_mesh,
    out_type=jax.ShapeDtypeStruct((NC, N_NODES, 16), jnp.float32),
    scratch_types=[
        pltpu.VMEM((CH_DEG, 16), jnp.float32),
        pltpu.VMEM((EPW,), jnp.int32),
        pltpu.VMEM_SHARED((N_NODES, 16), jnp.float32),
        pltpu.SemaphoreType.DMA,
        pltpu.SemaphoreType.DMA,
    ],
)
def _deg_kernel(dst_hbm, ones_hbm, out_hbm, ones_v, idx_v, acc_sh, isem,
                ssem):
    cid = lax.axis_index("c")
    sid = lax.axis_index("s")
    wid = sid * NC + cid

    pltpu.async_copy(dst_hbm.at[pl.ds(wid * EPW, EPW)], idx_v, isem)

    r0 = sid * ROWS_PER_TILE
    pltpu.sync_copy(ones_hbm.at[pl.ds(r0, ROWS_PER_TILE)],
                    acc_sh.at[pl.ds(r0, ROWS_PER_TILE)])

    @pl.when(sid == 0)
    def _():
        pltpu.sync_copy(ones_hbm.at[pl.ds(TAIL_OFF, TAIL_ROWS)],
                        acc_sh.at[pl.ds(TAIL_OFF, TAIL_ROWS)])

    pltpu.sync_copy(ones_hbm.at[pl.ds(0, CH_DEG)], ones_v)
    _drain(dst_hbm.at[pl.ds(0, EPW)], idx_v, isem)
    plsc.subcore_barrier()

    @pl.loop(0, N_CHUNKS_DEG)
    def _(i):
        pltpu.sync_copy(ones_v,
                        acc_sh.at[idx_v.at[pl.ds(i * CH_DEG, CH_DEG)]],
                        add=True)

    plsc.subcore_barrier()

    pltpu.sync_copy(acc_sh.at[pl.ds(r0, ROWS_PER_TILE)],
                    out_hbm.at[cid, pl.ds(r0, ROWS_PER_TILE)])

    @pl.when(sid == 0)
    def _():
        pltpu.sync_copy(acc_sh.at[pl.ds(TAIL_OFF, TAIL_ROWS)],
                        out_hbm.at[cid, pl.ds(TAIL_OFF, TAIL_ROWS)])


@functools.partial(
    pl.kernel,
    mesh=_mesh,
    out_type=jax.ShapeDtypeStruct((NC, N_NODES, D), jnp.float32),
    scratch_types=[
        pltpu.VMEM((EPW,), jnp.int32),
        pltpu.VMEM((EPW,), jnp.int32),
        pltpu.VMEM((CH, D), jnp.float32),
        pltpu.VMEM((CH, D), jnp.float32),
        pltpu.VMEM_SHARED((N_NODES, D), jnp.float32),
        pltpu.SemaphoreType.DMA,
        pltpu.SemaphoreType.DMA,
        pltpu.SemaphoreType.DMA,
        pltpu.SemaphoreType.DMA,
        pltpu.SemaphoreType.DMA,
    ],
)
def _segsum_kernel(y_hbm, src_hbm, dst_hbm, out_hbm,
                   src_v, dst_v, rows_a, rows_b, acc_sh,
                   isem, gsem_a, gsem_b, ssem_a, ssem_b):
    cid = lax.axis_index("c")
    sid = lax.axis_index("s")
    wid = sid * NC + cid
    base = wid * EPW

    pltpu.async_copy(src_hbm.at[pl.ds(base, EPW)], src_v, isem)
    pltpu.async_copy(dst_hbm.at[pl.ds(base, EPW)], dst_v, isem)

    r0 = sid * ROWS_PER_TILE
    pltpu.sync_copy(y_hbm.at[pl.ds(r0, ROWS_PER_TILE)],
                    acc_sh.at[pl.ds(r0, ROWS_PER_TILE)])

    @pl.when(sid == 0)
    def _():
        pltpu.sync_copy(y_hbm.at[pl.ds(TAIL_OFF, TAIL_ROWS)],
                        acc_sh.at[pl.ds(TAIL_OFF, TAIL_ROWS)])

    _drain(src_hbm.at[pl.ds(0, EPW)], src_v, isem)
    _drain(dst_hbm.at[pl.ds(0, EPW)], dst_v, isem)
    plsc.subcore_barrier()

    rows = (rows_a, rows_b)
    gsem = (gsem_a, gsem_b)
    ssem = (ssem_a, ssem_b)

    def issue_gather(j, b):
        pltpu.async_copy(y_hbm.at[src_v.at[pl.ds(j * CH, CH)]], rows[b],
                         gsem[b])

    issue_gather(0, 0)

    @pl.loop(0, N_CHUNKS)
    def _(i):
        for b in range(2):
            @pl.when(lax.rem(i, 2) == b)
            def _():
                @pl.when(i >= 1)
                def _():
                    _drain(y_hbm.at[pl.ds(0, CH)], rows[1 - b], ssem[1 - b])

                @pl.when(i + 1 < N_CHUNKS)
                def _():
                    issue_gather(i + 1, 1 - b)

                _drain(y_hbm.at[pl.ds(0, CH)], rows[b], gsem[b])
                pltpu.async_copy(rows[b],
                                 acc_sh.at[dst_v.at[pl.ds(i * CH, CH)]],
                                 ssem[b], add=True)

    _drain(y_hbm.at[pl.ds(0, CH)], rows[(N_CHUNKS - 1) % 2],
           ssem[(N_CHUNKS - 1) % 2])
    plsc.subcore_barrier()

    pltpu.sync_copy(acc_sh.at[pl.ds(r0, ROWS_PER_TILE)],
                    out_hbm.at[cid, pl.ds(r0, ROWS_PER_TILE)])

    @pl.when(sid == 0)
    def _():
        pltpu.sync_copy(acc_sh.at[pl.ds(TAIL_OFF, TAIL_ROWS)],
                        out_hbm.at[cid, pl.ds(TAIL_OFF, TAIL_ROWS)])


R = 2000


def _mm_scale_body(c_ref, x_ref, w_ref, dis_ref, y_ref):
    xw = lax.dot_general(x_ref[...], w_ref[...],
                         (((1,), (0,)), ((), ())),
                         preferred_element_type=jnp.float32,
                         precision=lax.Precision.HIGHEST)
    deg = c_ref[0, :, 0:1] + c_ref[1, :, 0:1] - 1.0
    dis = lax.rsqrt(deg)
    dis_ref[...] = dis
    y_ref[...] = xw * dis


def _mm_scale(counts, x, w):
    grid = (N_NODES // R,)
    return pl.pallas_call(
        _mm_scale_body,
        grid=grid,
        in_specs=[
            pl.BlockSpec((NC, R, 16), lambda i: (0, i, 0)),
            pl.BlockSpec((R, D), lambda i: (i, 0)),
            pl.BlockSpec((D, D), lambda i: (0, 0)),
        ],
        out_specs=[
            pl.BlockSpec((R, 1), lambda i: (i, 0)),
            pl.BlockSpec((R, D), lambda i: (i, 0)),
        ],
        out_shape=[
            jax.ShapeDtypeStruct((N_NODES, 1), jnp.float32),
            jax.ShapeDtypeStruct((N_NODES, D), jnp.float32),
        ],
    )(counts, x, w)


def _mid_body(s_ref, y1_ref, dis_ref, b_ref, w_ref, y2_ref):
    dis = dis_ref[...]
    comb = dis * (s_ref[0] + s_ref[1] - y1_ref[...]) + b_ref[...]
    h = jnp.maximum(comb, 0.0)
    hw = lax.dot_general(h, w_ref[...], (((1,), (0,)), ((), ())),
                         preferred_element_type=jnp.float32,
                         precision=lax.Precision.HIGHEST)
    y2_ref[...] = hw * dis


def _mid(s, y1, dis, b, w):
    grid = (N_NODES // R,)
    return pl.pallas_call(
        _mid_body,
        grid=grid,
        in_specs=[
            pl.BlockSpec((NC, R, D), lambda i: (0, i, 0)),
            pl.BlockSpec((R, D), lambda i: (i, 0)),
            pl.BlockSpec((R, 1), lambda i: (i, 0)),
            pl.BlockSpec((D,), lambda i: (0,)),
            pl.BlockSpec((D, D), lambda i: (0, 0)),
        ],
        out_specs=pl.BlockSpec((R, D), lambda i: (i, 0)),
        out_shape=jax.ShapeDtypeStruct((N_NODES, D), jnp.float32),
    )(s, y1, dis, b, w)


def _post_body(s_ref, y2_ref, dis_ref, b_ref, out_ref):
    dis = dis_ref[...]
    out_ref[...] = dis * (s_ref[0] + s_ref[1] - y2_ref[...]) + b_ref[...]


def _post(s, y2, dis, b):
    grid = (N_NODES // R,)
    return pl.pallas_call(
        _post_body,
        grid=grid,
        in_specs=[
            pl.BlockSpec((NC, R, D), lambda i: (0, i, 0)),
            pl.BlockSpec((R, D), lambda i: (i, 0)),
            pl.BlockSpec((R, 1), lambda i: (i, 0)),
            pl.BlockSpec((D,), lambda i: (0,)),
        ],
        out_specs=pl.BlockSpec((R, D), lambda i: (i, 0)),
        out_shape=jax.ShapeDtypeStruct((N_NODES, D), jnp.float32),
    )(s, y2, dis, b)


@jax.jit
def _run(x, ei, W1, b1, W2, b2):
    src = ei[0].astype(jnp.int32)
    dst = ei[1].astype(jnp.int32)
    ones = jnp.ones((N_NODES, 16), jnp.float32)

    counts = _deg_kernel(dst, ones)
    dis, y1 = _mm_scale(counts, x, W1)
    s1 = _segsum_kernel(y1, src, dst)
    y2 = _mid(s1, y1, dis, b1, W2)
    s2 = _segsum_kernel(y2, src, dst)
    out = _post(s2, y2, dis, b2)
    return out


def kernel(x, ei, W1, b1, W2, b2):
    return _run(x, ei, W1, b1, W2, b2)


# speedup vs baseline: 1.1359x; 1.1359x over previous
import functools

import jax
import jax.numpy as jnp
from jax import lax
from jax.experimental import pallas as pl
from jax.experimental.pallas import tpu as pltpu
from jax.experimental.pallas import tpu_sc as plsc

N_NODES = 10000
N_EDGES = 320000
D = 128

NC = 2
NS = 16
NW = NC * NS
EPW = N_EDGES // NW

CH = 80
N_CHUNKS = EPW // CH

CH_DEG = 200
N_CHUNKS_DEG = EPW // CH_DEG


def _drain(dummy_hbm_slice, dst_ref, sem):
    pltpu.make_async_copy(dummy_hbm_slice, dst_ref, sem).wait()

ROWS_PER_TILE = 624
TAIL_ROWS = N_NODES - NS * ROWS_PER_TILE
TAIL_OFF = NS * ROWS_PER_TILE

_mesh = plsc.VectorSubcoreMesh(core_axis_name="c", subcore_axis_name="s")


@functools.partial(
    pl.kernel,
    mesh=_mesh,
    out_type=jax.ShapeDtypeStruct((NC, N_NODES, 16), jnp.float32),
    scratch_types=[
        pltpu.VMEM((CH_DEG, 16), jnp.float32),
        pltpu.VMEM((EPW,), jnp.int32),
        pltpu.VMEM_SHARED((N_NODES, 16), jnp.float32),
        pltpu.SemaphoreType.DMA,
        pltpu.SemaphoreType.DMA,
    ],
)
def _deg_kernel(dst_hbm, ones_hbm, out_hbm, ones_v, idx_v, acc_sh, isem,
                ssem):
    cid = lax.axis_index("c")
    sid = lax.axis_index("s")
    wid = sid * NC + cid

    pltpu.async_copy(dst_hbm.at[pl.ds(wid * EPW, EPW)], idx_v, isem)

    r0 = sid * ROWS_PER_TILE
    pltpu.sync_copy(ones_hbm.at[pl.ds(r0, ROWS_PER_TILE)],
                    acc_sh.at[pl.ds(r0, ROWS_PER_TILE)])

    @pl.when(sid == 0)
    def _():
        pltpu.sync_copy(ones_hbm.at[pl.ds(TAIL_OFF, TAIL_ROWS)],
                        acc_sh.at[pl.ds(TAIL_OFF, TAIL_ROWS)])

    pltpu.sync_copy(ones_hbm.at[pl.ds(0, CH_DEG)], ones_v)
    _drain(dst_hbm.at[pl.ds(0, EPW)], idx_v, isem)
    plsc.subcore_barrier()

    @pl.loop(0, N_CHUNKS_DEG)
    def _(i):
        pltpu.sync_copy(ones_v,
                        acc_sh.at[idx_v.at[pl.ds(i * CH_DEG, CH_DEG)]],
                        add=True)

    plsc.subcore_barrier()

    pltpu.sync_copy(acc_sh.at[pl.ds(r0, ROWS_PER_TILE)],
                    out_hbm.at[cid, pl.ds(r0, ROWS_PER_TILE)])

    @pl.when(sid == 0)
    def _():
        pltpu.sync_copy(acc_sh.at[pl.ds(TAIL_OFF, TAIL_ROWS)],
                        out_hbm.at[cid, pl.ds(TAIL_OFF, TAIL_ROWS)])


@functools.partial(
    pl.kernel,
    mesh=_mesh,
    out_type=jax.ShapeDtypeStruct((NC, N_NODES, D), jnp.float32),
    scratch_types=[
        pltpu.VMEM((EPW,), jnp.int32),
        pltpu.VMEM((EPW,), jnp.int32),
        pltpu.VMEM((CH, D), jnp.float32),
        pltpu.VMEM((CH, D), jnp.float32),
        pltpu.VMEM((CH, D), jnp.float32),
        pltpu.VMEM_SHARED((N_NODES, D), jnp.float32),
        pltpu.SemaphoreType.DMA,
        pltpu.SemaphoreType.DMA,
        pltpu.SemaphoreType.DMA,
        pltpu.SemaphoreType.DMA,
        pltpu.SemaphoreType.DMA,
        pltpu.SemaphoreType.DMA,
        pltpu.SemaphoreType.DMA,
    ],
)
def _segsum_kernel(y_hbm, src_hbm, dst_hbm, out_hbm,
                   src_v, dst_v, rows_a, rows_b, rows_c, acc_sh,
                   isem, gsem_a, gsem_b, gsem_c, ssem_a, ssem_b, ssem_c):
    cid = lax.axis_index("c")
    sid = lax.axis_index("s")
    wid = sid * NC + cid
    base = wid * EPW

    pltpu.async_copy(src_hbm.at[pl.ds(base, EPW)], src_v, isem)
    pltpu.async_copy(dst_hbm.at[pl.ds(base, EPW)], dst_v, isem)

    r0 = sid * ROWS_PER_TILE
    pltpu.sync_copy(y_hbm.at[pl.ds(r0, ROWS_PER_TILE)],
                    acc_sh.at[pl.ds(r0, ROWS_PER_TILE)])

    @pl.when(sid == 0)
    def _():
        pltpu.sync_copy(y_hbm.at[pl.ds(TAIL_OFF, TAIL_ROWS)],
                        acc_sh.at[pl.ds(TAIL_OFF, TAIL_ROWS)])

    _drain(src_hbm.at[pl.ds(0, EPW)], src_v, isem)
    _drain(dst_hbm.at[pl.ds(0, EPW)], dst_v, isem)
    plsc.subcore_barrier()

    rows = (rows_a, rows_b, rows_c)
    gsem = (gsem_a, gsem_b, gsem_c)
    ssem = (ssem_a, ssem_b, ssem_c)

    def issue_gather(j, b):
        pltpu.async_copy(y_hbm.at[src_v.at[pl.ds(j * CH, CH)]], rows[b],
                         gsem[b])

    issue_gather(0, 0)
    issue_gather(1, 1)

    @pl.loop(0, N_CHUNKS)
    def _(i):
        for b in range(3):
            @pl.when(lax.rem(i, 3) == b)
            def _():
                bp = (b + 2) % 3

                _drain(y_hbm.at[pl.ds(0, CH)], rows[b], gsem[b])

                @pl.when(i >= 1)
                def _():
                    _drain(y_hbm.at[pl.ds(0, CH)], rows[bp], ssem[bp])

                @pl.when(i + 2 < N_CHUNKS)
                def _():
                    issue_gather(i + 2, bp)

                pltpu.async_copy(rows[b],
                                 acc_sh.at[dst_v.at[pl.ds(i * CH, CH)]],
                                 ssem[b], add=True)

    _drain(y_hbm.at[pl.ds(0, CH)], rows[(N_CHUNKS - 1) % 3],
           ssem[(N_CHUNKS - 1) % 3])
    plsc.subcore_barrier()

    pltpu.sync_copy(acc_sh.at[pl.ds(r0, ROWS_PER_TILE)],
                    out_hbm.at[cid, pl.ds(r0, ROWS_PER_TILE)])

    @pl.when(sid == 0)
    def _():
        pltpu.sync_copy(acc_sh.at[pl.ds(TAIL_OFF, TAIL_ROWS)],
                        out_hbm.at[cid, pl.ds(TAIL_OFF, TAIL_ROWS)])


R = 2000


def _mm_scale_body(c_ref, x_ref, w_ref, dis_ref, y_ref):
    xw = lax.dot_general(x_ref[...], w_ref[...],
                         (((1,), (0,)), ((), ())),
                         preferred_element_type=jnp.float32,
                         precision=lax.Precision.HIGHEST)
    deg = c_ref[0, :, 0:1] + c_ref[1, :, 0:1] - 1.0
    dis = lax.rsqrt(deg)
    dis_ref[...] = dis
    y_ref[...] = xw * dis


def _mm_scale(counts, x, w):
    grid = (N_NODES // R,)
    return pl.pallas_call(
        _mm_scale_body,
        grid=grid,
        in_specs=[
            pl.BlockSpec((NC, R, 16), lambda i: (0, i, 0)),
            pl.BlockSpec((R, D), lambda i: (i, 0)),
            pl.BlockSpec((D, D), lambda i: (0, 0)),
        ],
        out_specs=[
            pl.BlockSpec((R, 1), lambda i: (i, 0)),
            pl.BlockSpec((R, D), lambda i: (i, 0)),
        ],
        out_shape=[
            jax.ShapeDtypeStruct((N_NODES, 1), jnp.float32),
            jax.ShapeDtypeStruct((N_NODES, D), jnp.float32),
        ],
    )(counts, x, w)


def _mid_body(s_ref, y1_ref, dis_ref, b_ref, w_ref, y2_ref):
    dis = dis_ref[...]
    comb = dis * (s_ref[0] + s_ref[1] - y1_ref[...]) + b_ref[...]
    h = jnp.maximum(comb, 0.0)
    hw = lax.dot_general(h, w_ref[...], (((1,), (0,)), ((), ())),
                         preferred_element_type=jnp.float32,
                         precision=lax.Precision.HIGHEST)
    y2_ref[...] = hw * dis


def _mid(s, y1, dis, b, w):
    grid = (N_NODES // R,)
    return pl.pallas_call(
        _mid_body,
        grid=grid,
        in_specs=[
            pl.BlockSpec((NC, R, D), lambda i: (0, i, 0)),
            pl.BlockSpec((R, D), lambda i: (i, 0)),
            pl.BlockSpec((R, 1), lambda i: (i, 0)),
            pl.BlockSpec((D,), lambda i: (0,)),
            pl.BlockSpec((D, D), lambda i: (0, 0)),
        ],
        out_specs=pl.BlockSpec((R, D), lambda i: (i, 0)),
        out_shape=jax.ShapeDtypeStruct((N_NODES, D), jnp.float32),
    )(s, y1, dis, b, w)


def _post_body(s_ref, y2_ref, dis_ref, b_ref, out_ref):
    dis = dis_ref[...]
    out_ref[...] = dis * (s_ref[0] + s_ref[1] - y2_ref[...]) + b_ref[...]


def _post(s, y2, dis, b):
    grid = (N_NODES // R,)
    return pl.pallas_call(
        _post_body,
        grid=grid,
        in_specs=[
            pl.BlockSpec((NC, R, D), lambda i: (0, i, 0)),
            pl.BlockSpec((R, D), lambda i: (i, 0)),
            pl.BlockSpec((R, 1), lambda i: (i, 0)),
            pl.BlockSpec((D,), lambda i: (0,)),
        ],
        out_specs=pl.BlockSpec((R, D), lambda i: (i, 0)),
        out_shape=jax.ShapeDtypeStruct((N_NODES, D), jnp.float32),
    )(s, y2, dis, b)


@jax.jit
def _run(x, ei, W1, b1, W2, b2):
    src = ei[0].astype(jnp.int32)
    dst = ei[1].astype(jnp.int32)
    ones = jnp.ones((N_NODES, 16), jnp.float32)

    counts = _deg_kernel(dst, ones)
    dis, y1 = _mm_scale(counts, x, W1)
    s1 = _segsum_kernel(y1, src, dst)
    y2 = _mid(s1, y1, dis, b1, W2)
    s2 = _segsum_kernel(y2, src, dst)
    out = _post(s2, y2, dis, b2)
    return out


def kernel(x, ei, W1, b1, W2, b2):
    return _run(x, ei, W1, b1, W2, b2)
